# baseline (speedup 1.0000x reference)
import functools
import math

import jax
import jax.numpy as jnp
from jax.experimental import pallas as pl
from jax.experimental.pallas import tpu as pltpu

B, SMAX, NS, H = 2, 2048, 1024, 16
D = NS // H
S = SMAX
SCALE = 1.0 / math.sqrt(D)
NEG_INF = float(jnp.finfo(jnp.float32).min)


_ROW_BLK = 512
_NSB = S // _ROW_BLK


def _qkv_kernel(x_ref, w_ref, b_ref, q_ref, k_ref, v_ref):
    x = x_ref[...]
    outs = (q_ref, k_ref, v_ref)
    for g in range(12):
        w = w_ref[:, g * 256:(g + 1) * 256]
        pr = jnp.dot(x, w, preferred_element_type=jnp.float32)
        pr = pr + b_ref[:, g * 256:(g + 1) * 256]
        tgt = outs[g // 4]
        for i in range(4):
            h = (g % 4) * 4 + i
            tgt[0, h] = pr[:, i * 64:(i + 1) * 64]


def _qkv_proj(x2d, w_cat, b_cat):
    grid = (x2d.shape[0] // _ROW_BLK,)
    bhsd = jax.ShapeDtypeStruct((B, H, S, D), jnp.float32)
    out_spec = pl.BlockSpec((1, H, _ROW_BLK, D),
                            lambda r: (r // _NSB, 0, r % _NSB, 0))
    return pl.pallas_call(
        _qkv_kernel,
        grid=grid,
        in_specs=[
            pl.BlockSpec((_ROW_BLK, NS), lambda r: (r, 0)),
            pl.BlockSpec((NS, 3 * NS), lambda r: (0, 0)),
            pl.BlockSpec((1, 3 * NS), lambda r: (0, 0)),
        ],
        out_specs=[out_spec, out_spec, out_spec],
        out_shape=[bhsd, bhsd, bhsd],
        compiler_params=pltpu.CompilerParams(
            dimension_semantics=("parallel",),
            vmem_limit_bytes=56 * 1024 * 1024,
        ),
        name="qkv_proj",
    )(x2d, w_cat, b_cat)


_BQ = 256
_BK = 256
_NQ = S // _BQ


def _attn_kernel(q_ref, k_ref, v_ref, m_ref, o_ref):
    qi = pl.program_id(1)
    q = q_ref[0, 0] * SCALE
    row_ids = qi * _BQ + jax.lax.broadcasted_iota(jnp.int32, (_BQ, _BK), 0)

    def body(j, carry):
        m_prev, l_prev, acc = carry
        off = pl.multiple_of(j * _BK, _BK)
        k = k_ref[0, 0, pl.ds(off, _BK), :]
        v = v_ref[0, 0, pl.ds(off, _BK), :]
        s = jax.lax.dot_general(q, k, (((1,), (1,)), ((), ())),
                                preferred_element_type=jnp.float32)
        col_ids = j * _BK + jax.lax.broadcasted_iota(jnp.int32, (_BQ, _BK), 1)
        keyv = m_ref[0, 0, pl.ds(off, _BK)]
        valid = jnp.logical_and(row_ids >= col_ids, keyv[None, :] > 0.0)
        s = jnp.where(valid, s, NEG_INF)
        m_cur = jnp.max(s, axis=-1, keepdims=True)
        m_next = jnp.maximum(m_prev, m_cur)
        alpha = jnp.exp(m_prev - m_next)
        p = jnp.exp(s - m_next[:, :1])
        l_next = alpha * l_prev + jnp.sum(p, axis=-1, keepdims=True)
        acc = acc * alpha[:, :1] + jax.lax.dot_general(
            p, v, (((1,), (0,)), ((), ())),
            preferred_element_type=jnp.float32)
        return m_next, l_next, acc

    m0 = jnp.full((_BQ, 128), NEG_INF, jnp.float32)
    l0 = jnp.zeros((_BQ, 128), jnp.float32)
    acc0 = jnp.zeros((_BQ, D), jnp.float32)
    m_fin, l_fin, acc = jax.lax.fori_loop(0, qi + 1, body, (m0, l0, acc0))
    o_ref[0, 0] = acc / l_fin[:, :1]


def _attention(q, kc, vc, mask3):
    grid = (B * H, _NQ)
    kv_spec = pl.BlockSpec((1, 1, S, D),
                           lambda bh, qi: (bh // H, bh % H, 0, 0))
    return pl.pallas_call(
        _attn_kernel,
        grid=grid,
        in_specs=[
            pl.BlockSpec((1, 1, _BQ, D),
                         lambda bh, qi: (bh // H, bh % H, qi, 0)),
            kv_spec,
            kv_spec,
            pl.BlockSpec((1, 1, SMAX), lambda bh, qi: (bh // H, 0, 0)),
        ],
        out_specs=pl.BlockSpec((1, 1, _BQ, D),
                               lambda bh, qi: (bh // H, bh % H, qi, 0)),
        out_shape=jax.ShapeDtypeStruct((B, H, S, D), jnp.float32),
        compiler_params=pltpu.CompilerParams(
            dimension_semantics=("parallel", "arbitrary"),
            vmem_limit_bytes=32 * 1024 * 1024,
        ),
        name="flash_attn",
    )(q, kc, vc, mask3)


def _out_kernel(a_ref, w_ref, b_ref, o_ref):
    xb = jnp.concatenate([a_ref[0, h] for h in range(H)], axis=-1)
    for g in range(4):
        w = w_ref[:, g * 256:(g + 1) * 256]
        pr = jnp.dot(xb, w, preferred_element_type=jnp.float32)
        o_ref[0, :, g * 256:(g + 1) * 256] = pr + b_ref[:, g * 256:(g + 1) * 256]


def _out_proj(ao, w_t, b2d):
    grid = (B * _NSB,)
    return pl.pallas_call(
        _out_kernel,
        grid=grid,
        in_specs=[
            pl.BlockSpec((1, H, _ROW_BLK, D),
                         lambda r: (r // _NSB, 0, r % _NSB, 0)),
            pl.BlockSpec((NS, NS), lambda r: (0, 0)),
            pl.BlockSpec((1, NS), lambda r: (0, 0)),
        ],
        out_specs=pl.BlockSpec((1, _ROW_BLK, NS),
                               lambda r: (r // _NSB, r % _NSB, 0)),
        out_shape=jax.ShapeDtypeStruct((B, S, NS), jnp.float32),
        compiler_params=pltpu.CompilerParams(
            dimension_semantics=("parallel",),
            vmem_limit_bytes=48 * 1024 * 1024,
        ),
        name="out_proj",
    )(ao, w_t, b2d)


def kernel(x, mask, Wq, bq, Wk, bk, Wv, bv, Wo, bo, cache_k, cache_v):
    x2d = x.reshape(B * S, NS)
    w_cat = jnp.concatenate([Wq.T, Wk.T, Wv.T], axis=1)
    b_cat = jnp.concatenate([bq, bk, bv]).reshape(1, 3 * NS)
    q, kc, vc = _qkv_proj(x2d, w_cat, b_cat)
    ao = _attention(q, kc, vc, mask.reshape(B, 1, SMAX))
    out = _out_proj(ao, Wo.T, bo.reshape(1, NS))
    return (out, kc, vc)
